# baseline (device time: 20729 ns/iter reference)
import jax
import jax.numpy as jnp
from jax import lax
from jax.experimental import pallas as pl
from jax.experimental.pallas import tpu as pltpu

N_DEV = 4


def kernel(x):
    m, n = x.shape
    q = m // 4
    h = m // 2
    cw = n // 2

    CA = pl.ds(0, cw)
    CB = pl.ds(cw, cw)

    def body(x_ref, out_ref, st1a, st1b, r1a, r1b, r2a, r2b,
             send_sems, recv_sems):
        my = lax.axis_index("i")
        g = my // 2
        b = lax.rem(lax.rem(my, 2) + g, 2)
        p_b = jnp.bitwise_xor(my, 1)
        p_g = jnp.bitwise_xor(my, 3)

        def rows(k):
            return pl.ds(k * q, q)

        def bf(row_slice, col_slice):
            return x_ref[row_slice, col_slice].astype(jnp.bfloat16)

        barrier_sem = pltpu.get_barrier_semaphore()
        for nbr in (p_b, p_g):
            pl.semaphore_signal(
                barrier_sem, inc=1,
                device_id=(nbr,), device_id_type=pl.DeviceIdType.MESH,
            )
        pl.semaphore_wait(barrier_sem, 2)

        def rdma(src, dst, sem_idx, dev):
            return pltpu.make_async_remote_copy(
                src_ref=src, dst_ref=dst,
                send_sem=send_sems.at[sem_idx],
                recv_sem=recv_sems.at[sem_idx],
                device_id=(dev,), device_id_type=pl.DeviceIdType.MESH,
            )

        hk = pl.ds(g * h, h)
        ha = pl.ds((1 - g) * h, h)
        st1a[:, :] = bf(ha, CA)
        st1b[0, :, :] = bf(rows(1 - b), CB)
        st1b[1, :, :] = bf(rows(3 - b), CB)
        c1a = rdma(st1a, r1a, 0, p_g)
        c1b = rdma(st1b, r1b, 1, p_b)
        c1a.start()
        c1b.start()
        out_ref[hk, CA] = bf(hk, CA)
        out_ref[rows(b), CB] = bf(rows(b), CB)
        out_ref[rows(2 + b), CB] = bf(rows(2 + b), CB)
        c1a.wait()
        c1b.wait()
        out_ref[hk, CA] = out_ref[hk, CA] + r1a[:, :]
        out_ref[rows(b), CB] = out_ref[rows(b), CB] + r1b[0, :, :]
        out_ref[rows(2 + b), CB] = out_ref[rows(2 + b), CB] + r1b[1, :, :]

        k_own = 2 * g + b
        k_pb = 2 * g + 1 - b
        k_pg = 2 * (1 - g) + b
        c2a = rdma(out_ref.at[rows(k_pb), CA], r2a, 2, p_b)
        c2b = rdma(out_ref.at[rows(k_pg), CB], r2b, 3, p_g)
        c2a.start()
        c2b.start()

        c2a.wait()
        out_ref[rows(k_own), CA] = out_ref[rows(k_own), CA] + r2a[:, :]
        c3a = rdma(out_ref.at[rows(k_own), CA],
                   out_ref.at[rows(k_own), CA], 4, p_b)
        c4ao = rdma(out_ref.at[rows(k_own), CA],
                    out_ref.at[rows(k_own), CA], 6, p_g)
        c3a.start()
        c4ao.start()
        c2b.wait()
        out_ref[rows(k_own), CB] = out_ref[rows(k_own), CB] + r2b[:, :]
        c3b = rdma(out_ref.at[rows(k_own), CB],
                   out_ref.at[rows(k_own), CB], 5, p_g)
        c4bo = rdma(out_ref.at[rows(k_own), CB],
                    out_ref.at[rows(k_own), CB], 7, p_b)
        c3b.start()
        c4bo.start()
        c3a.wait()
        c4ar = rdma(out_ref.at[rows(k_pb), CA],
                    out_ref.at[rows(k_pb), CA], 8, p_g)
        c4ar.start()
        c3b.wait()
        c4br = rdma(out_ref.at[rows(k_pg), CB],
                    out_ref.at[rows(k_pg), CB], 9, p_b)
        c4br.start()
        c4ao.wait()
        c4bo.wait()
        c4ar.wait()
        c4br.wait()

    return pl.pallas_call(
        body,
        out_shape=jax.ShapeDtypeStruct((m, n), jnp.bfloat16),
        in_specs=[pl.BlockSpec(memory_space=pltpu.VMEM)],
        out_specs=pl.BlockSpec(memory_space=pltpu.VMEM),
        scratch_shapes=[
            pltpu.VMEM((h, cw), jnp.bfloat16),
            pltpu.VMEM((2, q, cw), jnp.bfloat16),
            pltpu.VMEM((h, cw), jnp.bfloat16),
            pltpu.VMEM((2, q, cw), jnp.bfloat16),
            pltpu.VMEM((q, cw), jnp.bfloat16),
            pltpu.VMEM((q, cw), jnp.bfloat16),
            pltpu.SemaphoreType.DMA((10,)),
            pltpu.SemaphoreType.DMA((10,)),
        ],
        compiler_params=pltpu.CompilerParams(collective_id=0),
    )(x)


# device time: 18812 ns/iter; 1.1019x vs baseline; 1.1019x over previous
import jax
import jax.numpy as jnp
from jax import lax
from jax.experimental import pallas as pl
from jax.experimental.pallas import tpu as pltpu

N_DEV = 4


def kernel(x):
    m, n = x.shape
    q = m // 4
    cw = n // 2

    CA = pl.ds(0, cw)
    CB = pl.ds(cw, cw)

    def body(x_ref, out_ref, st, ra, rb, send_sems, recv_sems):
        my = lax.axis_index("i")
        g = my // 2
        b = lax.rem(lax.rem(my, 2) + g, 2)
        p_b = jnp.bitwise_xor(my, 1)
        p_g = jnp.bitwise_xor(my, 3)
        diag = jnp.bitwise_xor(my, 2)

        peers = (p_b, p_g, diag)
        g_pb, b_pb = g, 1 - b
        g_pg, b_pg = 1 - g, b
        g_dg, b_dg = 1 - g, 1 - b
        peer_k = (2 * g_pb + b_pb, 2 * g_pg + b_pg, 2 * g_dg + b_dg)
        k_own = 2 * g + b

        def rows(k):
            return pl.ds(k * q, q)

        def bf(row_slice, col_slice):
            return x_ref[row_slice, col_slice].astype(jnp.bfloat16)

        barrier_sem = pltpu.get_barrier_semaphore()
        for nbr in (p_b, p_g):
            pl.semaphore_signal(
                barrier_sem, inc=1,
                device_id=(nbr,), device_id_type=pl.DeviceIdType.MESH,
            )
        pl.semaphore_wait(barrier_sem, 2)

        def rdma(src, dst, ch, dev):
            return pltpu.make_async_remote_copy(
                src_ref=src, dst_ref=dst,
                send_sem=send_sems.at[ch],
                recv_sem=recv_sems.at[ch],
                device_id=(dev,), device_id_type=pl.DeviceIdType.MESH,
            )

        rs_a = []
        for j in (2, 0, 1):
            st[j, :, :] = bf(rows(peer_k[j]), CA)
            c = rdma(st.at[j], ra.at[j], j, peers[j])
            c.start()
            rs_a.append(c)
        rs_b = []
        for j in (2, 0, 1):
            st[3 + j, :, :] = bf(rows(peer_k[j]), CB)
            c = rdma(st.at[3 + j], rb.at[j], 3 + j, peers[j])
            c.start()
            rs_b.append(c)
        out_ref[rows(k_own), :] = x_ref[rows(k_own), :].astype(jnp.bfloat16)

        for c in rs_a:
            c.wait()
        out_ref[rows(k_own), CA] = (
            out_ref[rows(k_own), CA]
            + ra[0, :, :] + ra[1, :, :] + ra[2, :, :])
        ag = []
        for j in (2, 0, 1):
            c = rdma(out_ref.at[rows(k_own), CA],
                     out_ref.at[rows(k_own), CA], 6 + j, peers[j])
            c.start()
            ag.append(c)

        for c in rs_b:
            c.wait()
        out_ref[rows(k_own), CB] = (
            out_ref[rows(k_own), CB]
            + rb[0, :, :] + rb[1, :, :] + rb[2, :, :])
        for j in (2, 0, 1):
            c = rdma(out_ref.at[rows(k_own), CB],
                     out_ref.at[rows(k_own), CB], 9 + j, peers[j])
            c.start()
            ag.append(c)

        for c in ag:
            c.wait()

    return pl.pallas_call(
        body,
        out_shape=jax.ShapeDtypeStruct((m, n), jnp.bfloat16),
        in_specs=[pl.BlockSpec(memory_space=pltpu.VMEM)],
        out_specs=pl.BlockSpec(memory_space=pltpu.VMEM),
        scratch_shapes=[
            pltpu.VMEM((6, q, cw), jnp.bfloat16),
            pltpu.VMEM((3, q, cw), jnp.bfloat16),
            pltpu.VMEM((3, q, cw), jnp.bfloat16),
            pltpu.SemaphoreType.DMA((12,)),
            pltpu.SemaphoreType.DMA((12,)),
        ],
        compiler_params=pltpu.CompilerParams(collective_id=0),
    )(x)


# device time: 18183 ns/iter; 1.1400x vs baseline; 1.0346x over previous
import jax
import jax.numpy as jnp
from jax import lax
from jax.experimental import pallas as pl
from jax.experimental.pallas import tpu as pltpu

N_DEV = 4
N_FLOWS = 4


def kernel(x):
    m, n = x.shape
    q = m // 4
    h = m // 2
    cw2 = n // N_FLOWS

    CA = pl.ds(0, n // 2)
    CB = pl.ds(n // 2, n // 2)

    def body(x_ref, out_ref, st1, r1, r2, send_sems, recv_sems):
        my = lax.axis_index("i")
        g = my // 2
        b = lax.rem(lax.rem(my, 2) + g, 2)
        p_b = jnp.bitwise_xor(my, 1)
        p_g = jnp.bitwise_xor(my, 3)

        k_own = 2 * g + b
        k_pb = 2 * g + 1 - b
        k_pg = 2 * (1 - g) + b
        hk = pl.ds(g * h, h)
        ha = pl.ds((1 - g) * h, h)

        def rows(k):
            return pl.ds(k * q, q)

        def bf(row_slice, col_slice):
            return x_ref[row_slice, col_slice].astype(jnp.bfloat16)

        barrier_sem = pltpu.get_barrier_semaphore()
        for nbr in (p_b, p_g):
            pl.semaphore_signal(
                barrier_sem, inc=1,
                device_id=(nbr,), device_id_type=pl.DeviceIdType.MESH,
            )
        pl.semaphore_wait(barrier_sem, 2)

        def rdma(src, dst, ch, dev):
            return pltpu.make_async_remote_copy(
                src_ref=src, dst_ref=dst,
                send_sem=send_sems.at[ch],
                recv_sem=recv_sems.at[ch],
                device_id=(dev,), device_id_type=pl.DeviceIdType.MESH,
            )

        flows = [
            ("A", pl.ds(0 * cw2, cw2)),
            ("B", pl.ds(2 * cw2, cw2)),
            ("A", pl.ds(1 * cw2, cw2)),
            ("B", pl.ds(3 * cw2, cw2)),
        ]

        def partners(kind):
            return (p_g, p_b) if kind == "A" else (p_b, p_g)

        def k_p2(kind):
            return k_pb if kind == "A" else k_pg

        c1 = []
        for f, (kind, C) in enumerate(flows):
            p1, _ = partners(kind)
            if kind == "A":
                st1[f, :, :] = bf(ha, C)
            else:
                st1[f, 0:q, :] = bf(rows(1 - b), C)
                st1[f, q:h, :] = bf(rows(3 - b), C)
            c = rdma(st1.at[f], r1.at[f], 5 * f + 0, p1)
            c.start()
            c1.append(c)
        out_ref[hk, CA] = bf(hk, CA)
        out_ref[rows(b), CB] = bf(rows(b), CB)
        out_ref[rows(2 + b), CB] = bf(rows(2 + b), CB)

        c2 = []
        for f, (kind, C) in enumerate(flows):
            _, p2 = partners(kind)
            c1[f].wait()
            if kind == "A":
                out_ref[hk, C] = out_ref[hk, C] + r1[f, :, :]
            else:
                out_ref[rows(b), C] = out_ref[rows(b), C] + r1[f, 0:q, :]
                out_ref[rows(2 + b), C] = (
                    out_ref[rows(2 + b), C] + r1[f, q:h, :])
            c = rdma(out_ref.at[rows(k_p2(kind)), C], r2.at[f],
                     5 * f + 1, p2)
            c.start()
            c2.append(c)

        c3, c4o = [], []
        for f, (kind, C) in enumerate(flows):
            p1, p2 = partners(kind)
            c2[f].wait()
            out_ref[rows(k_own), C] = out_ref[rows(k_own), C] + r2[f, :, :]
            ca = rdma(out_ref.at[rows(k_own), C],
                      out_ref.at[rows(k_own), C], 5 * f + 2, p2)
            cb = rdma(out_ref.at[rows(k_own), C],
                      out_ref.at[rows(k_own), C], 5 * f + 3, p1)
            ca.start()
            cb.start()
            c3.append(ca)
            c4o.append(cb)

        c4r = []
        for f, (kind, C) in enumerate(flows):
            p1, _ = partners(kind)
            c3[f].wait()
            c = rdma(out_ref.at[rows(k_p2(kind)), C],
                     out_ref.at[rows(k_p2(kind)), C], 5 * f + 4, p1)
            c.start()
            c4r.append(c)

        for f in range(N_FLOWS):
            c4o[f].wait()
            c4r[f].wait()

    return pl.pallas_call(
        body,
        out_shape=jax.ShapeDtypeStruct((m, n), jnp.bfloat16),
        in_specs=[pl.BlockSpec(memory_space=pltpu.VMEM)],
        out_specs=pl.BlockSpec(memory_space=pltpu.VMEM),
        scratch_shapes=[
            pltpu.VMEM((N_FLOWS, h, cw2), jnp.bfloat16),
            pltpu.VMEM((N_FLOWS, h, cw2), jnp.bfloat16),
            pltpu.VMEM((N_FLOWS, q, cw2), jnp.bfloat16),
            pltpu.SemaphoreType.DMA((5 * N_FLOWS,)),
            pltpu.SemaphoreType.DMA((5 * N_FLOWS,)),
        ],
        compiler_params=pltpu.CompilerParams(collective_id=0),
    )(x)


# device time: 18172 ns/iter; 1.1407x vs baseline; 1.0006x over previous
import jax
import jax.numpy as jnp
from jax import lax
from jax.experimental import pallas as pl
from jax.experimental.pallas import tpu as pltpu

N_DEV = 4


def kernel(x):
    m, n = x.shape
    q = m // 4
    h = m // 2
    o = q // 2
    cw = n // 2

    CA = pl.ds(0, cw)
    CB = pl.ds(cw, cw)

    def body(x_ref, out_ref, stA, stB, r1A, r1B, r2A, r2B,
             send_sems, recv_sems):
        my = lax.axis_index("i")
        g = my // 2
        b = lax.rem(lax.rem(my, 2) + g, 2)
        p_b = jnp.bitwise_xor(my, 1)
        p_g = jnp.bitwise_xor(my, 3)

        k_own = 2 * g + b
        k_pb = 2 * g + 1 - b
        k_pg = 2 * (1 - g) + b
        k_dg = 2 * (1 - g) + 1 - b
        hk = pl.ds(g * h, h)

        def oct_(k, j):
            return pl.ds(k * q + j * o, o)

        def bf(row_slice, col_slice):
            return x_ref[row_slice, col_slice].astype(jnp.bfloat16)

        barrier_sem = pltpu.get_barrier_semaphore()
        for nbr in (p_b, p_g):
            pl.semaphore_signal(
                barrier_sem, inc=1,
                device_id=(nbr,), device_id_type=pl.DeviceIdType.MESH,
            )
        pl.semaphore_wait(barrier_sem, 2)

        def rdma(src, dst, ch, dev):
            return pltpu.make_async_remote_copy(
                src_ref=src, dst_ref=dst,
                send_sem=send_sems.at[ch],
                recv_sem=recv_sems.at[ch],
                device_id=(dev,), device_id_type=pl.DeviceIdType.MESH,
            )


        c1A, c1B = [], []
        for j in (0, 1):
            stA[j, :, :] = bf(oct_(k_dg, j), CA)
            c = rdma(stA.at[j], r1A.at[j], 0 + j, p_g)
            c.start()
            c1A.append(c)
            stB[j, :, :] = bf(oct_(k_dg, j), CB)
            c = rdma(stB.at[j], r1B.at[j], 12 + j, p_b)
            c.start()
            c1B.append(c)
        for j in (2, 3):
            stA[j, :, :] = bf(oct_(k_pg, j - 2), CA)
            c = rdma(stA.at[j], r1A.at[j], 0 + j, p_g)
            c.start()
            c1A.append(c)
            stB[j, :, :] = bf(oct_(k_pb, j - 2), CB)
            c = rdma(stB.at[j], r1B.at[j], 12 + j, p_b)
            c.start()
            c1B.append(c)
        out_ref[hk, CA] = bf(hk, CA)
        rows_b = pl.ds(b * q, q)
        rows_2b = pl.ds((2 + b) * q, q)
        out_ref[rows_b, CB] = bf(rows_b, CB)
        out_ref[rows_2b, CB] = bf(rows_2b, CB)

        c2A, c2B = [], []
        for j in (0, 1):
            c1A[j].wait()
            out_ref[oct_(k_pb, j), CA] = (
                out_ref[oct_(k_pb, j), CA] + r1A[j, :, :])
            c = rdma(out_ref.at[oct_(k_pb, j), CA], r2A.at[j], 4 + j, p_b)
            c.start()
            c2A.append(c)
            c1B[j].wait()
            out_ref[oct_(k_pg, j), CB] = (
                out_ref[oct_(k_pg, j), CB] + r1B[j, :, :])
            c = rdma(out_ref.at[oct_(k_pg, j), CB], r2B.at[j], 16 + j, p_g)
            c.start()
            c2B.append(c)

        for j in (2, 3):
            c1A[j].wait()
            out_ref[oct_(k_own, j - 2), CA] = (
                out_ref[oct_(k_own, j - 2), CA] + r1A[j, :, :])
            c1B[j].wait()
            out_ref[oct_(k_own, j - 2), CB] = (
                out_ref[oct_(k_own, j - 2), CB] + r1B[j, :, :])

        c3A, c3B, c4oA, c4oB = [], [], [], []
        for j in (0, 1):
            c2A[j].wait()
            out_ref[oct_(k_own, j), CA] = (
                out_ref[oct_(k_own, j), CA] + r2A[j, :, :])
            ca = rdma(out_ref.at[oct_(k_own, j), CA],
                      out_ref.at[oct_(k_own, j), CA], 6 + j, p_b)
            cb = rdma(out_ref.at[oct_(k_own, j), CA],
                      out_ref.at[oct_(k_own, j), CA], 8 + j, p_g)
            ca.start()
            cb.start()
            c3A.append(ca)
            c4oA.append(cb)
            c2B[j].wait()
            out_ref[oct_(k_own, j), CB] = (
                out_ref[oct_(k_own, j), CB] + r2B[j, :, :])
            ca = rdma(out_ref.at[oct_(k_own, j), CB],
                      out_ref.at[oct_(k_own, j), CB], 18 + j, p_g)
            cb = rdma(out_ref.at[oct_(k_own, j), CB],
                      out_ref.at[oct_(k_own, j), CB], 20 + j, p_b)
            ca.start()
            cb.start()
            c3B.append(ca)
            c4oB.append(cb)

        c4rA, c4rB = [], []
        for j in (0, 1):
            c3A[j].wait()
            c = rdma(out_ref.at[oct_(k_pb, j), CA],
                     out_ref.at[oct_(k_pb, j), CA], 10 + j, p_g)
            c.start()
            c4rA.append(c)
            c3B[j].wait()
            c = rdma(out_ref.at[oct_(k_pg, j), CB],
                     out_ref.at[oct_(k_pg, j), CB], 22 + j, p_b)
            c.start()
            c4rB.append(c)

        for j in (0, 1):
            c4oA[j].wait()
            c4oB[j].wait()
            c4rA[j].wait()
            c4rB[j].wait()

    return pl.pallas_call(
        body,
        out_shape=jax.ShapeDtypeStruct((m, n), jnp.bfloat16),
        in_specs=[pl.BlockSpec(memory_space=pltpu.VMEM)],
        out_specs=pl.BlockSpec(memory_space=pltpu.VMEM),
        scratch_shapes=[
            pltpu.VMEM((4, o, cw), jnp.bfloat16),
            pltpu.VMEM((4, o, cw), jnp.bfloat16),
            pltpu.VMEM((4, o, cw), jnp.bfloat16),
            pltpu.VMEM((4, o, cw), jnp.bfloat16),
            pltpu.VMEM((2, o, cw), jnp.bfloat16),
            pltpu.VMEM((2, o, cw), jnp.bfloat16),
            pltpu.SemaphoreType.DMA((24,)),
            pltpu.SemaphoreType.DMA((24,)),
        ],
        compiler_params=pltpu.CompilerParams(collective_id=0),
    )(x)
